# baseline (speedup 1.0000x reference)
import jax
import jax.numpy as jnp
import numpy as np
from jax import lax
from jax.experimental import pallas as pl
from jax.experimental.pallas import tpu as pltpu
from jax.experimental.pallas import tpu_sc as plsc

B = 16384
F = 26
V = 100000
D = 32
NUM = 13
OUTW = F * D + NUM

NC = 2
NS = 16
NW = NC * NS
BPW = B // NW
C = 64
NSUB = BPW // C
GROUPS = C * F // 128


def _body(num_hbm, cat_hbm, ftab_hbm, off_hbm, out_hbm,
          idxraw_v, idxflat_v, off_v, emb_v, num_v, stage_v, sem):
    wid = lax.axis_index("s") * NC + lax.axis_index("c")
    lanes = lax.iota(jnp.int32, 16)
    num_mask = lanes < NUM

    pltpu.sync_copy(off_hbm, off_v)

    def chunk_body(j, _):
        row0 = wid * BPW + j * C

        pltpu.sync_copy(cat_hbm.at[pl.ds(row0 * F, C * F)], idxraw_v)
        pltpu.sync_copy(num_hbm.at[pl.ds(row0 * NUM, C * NUM)],
                        num_v.at[pl.ds(0, C * NUM)])

        def idx_body(g, _):
            for l in range(8):
                v = idxraw_v[pl.ds((g * 8 + l) * 16, 16)] \
                    + off_v[pl.ds((g * 8 + l) * 16, 16)]
                idxflat_v[g, pl.ds(l * 16, 16)] = v
            return 0

        lax.fori_loop(0, GROUPS, idx_body, 0)

        descs = []
        for g in range(GROUPS):
            descs.append(pltpu.async_copy(
                ftab_hbm.at[idxflat_v.at[g]],
                emb_v.at[pl.ds(g * 128, 128)], sem))
        for d in descs:
            d.wait()

        def row_body(c, _):
            sb = c * OUTW
            er = c * F
            for f in range(F):
                v0 = emb_v[er + f, pl.ds(0, 16)]
                v1 = emb_v[er + f, pl.ds(16, 16)]
                plsc.store_scatter(stage_v, [sb + f * D + lanes], v0)
                plsc.store_scatter(stage_v, [sb + f * D + 16 + lanes], v1)
            nv = plsc.load_gather(num_v, [c * NUM + lanes], mask=num_mask)
            plsc.store_scatter(stage_v, [sb + F * D + lanes], nv,
                               mask=num_mask)
            return 0

        lax.fori_loop(0, C, row_body, 0)

        pltpu.sync_copy(stage_v, out_hbm.at[pl.ds(row0 * OUTW, C * OUTW)])
        return 0

    lax.fori_loop(0, NSUB, chunk_body, 0)


@jax.jit
def _run(numeric, cat_indices, tables):
    ftab = tables.reshape(F * V, D)
    cat_flat = cat_indices.reshape(B * F)
    num_flat = numeric.reshape(B * NUM)
    off = jnp.asarray(np.tile(np.arange(F, dtype=np.int32) * V, C))

    mesh = plsc.VectorSubcoreMesh(core_axis_name="c", subcore_axis_name="s")
    k = pl.kernel(
        _body,
        out_type=jax.ShapeDtypeStruct((B * OUTW,), jnp.float32),
        mesh=mesh,
        scratch_types=[
            pltpu.VMEM((C * F,), jnp.int32),
            pltpu.VMEM((GROUPS, 128), jnp.int32),
            pltpu.VMEM((C * F,), jnp.int32),
            pltpu.VMEM((C * F, D), jnp.float32),
            pltpu.VMEM((C * NUM + 16,), jnp.float32),
            pltpu.VMEM((C * OUTW,), jnp.float32),
            pltpu.SemaphoreType.DMA,
        ],
        compiler_params=pltpu.CompilerParams(
            needs_layout_passes=False, use_tc_tiling_on_sc=False),
    )
    out_flat = k(num_flat, cat_flat, ftab, off)
    return out_flat.reshape(B, OUTW)


def kernel(numeric, cat_indices, tables):
    return _run(numeric, cat_indices, tables)
